# baseline (speedup 1.0000x reference)
import functools

import jax
import jax.numpy as jnp
from jax import lax
from jax.experimental import pallas as pl
from jax.experimental.pallas import tpu as pltpu
from jax.experimental.pallas import tpu_sc as plsc

NUM_USERS = 25000
NUM_ITEMS = 25000
N = NUM_USERS + NUM_ITEMS
D = 32
E = 1600000
NUM_LAYERS = 3
DROPOUT_P = 0.2

NC, NS, L = 2, 16, 16
N_PAD = 50176
NW = NC * NS
CH = 128
GROUP = 2
R_PER_W = 400
NGROUPS = R_PER_W // GROUP
NPAIRS = NGROUPS // 2
E_PAD = NW * R_PER_W * CH
ROWS_PER_TILE = N_PAD // NS
ZCH = 112
NZ = ROWS_PER_TILE // ZCH

_mesh = plsc.VectorSubcoreMesh(core_axis_name="c", subcore_axis_name="s")


@functools.partial(
    pl.kernel,
    out_type=(
        jax.ShapeDtypeStruct((N_PAD, D), jnp.float32),
        jax.ShapeDtypeStruct((N_PAD, D), jnp.float32),
    ),
    mesh=_mesh,
    compiler_params=pltpu.CompilerParams(use_tc_tiling_on_sc=False),
    scratch_types=dict(
        acc=pltpu.VMEM_SHARED((N_PAD, D), jnp.float32),
        src0=pltpu.VMEM((GROUP, CH), jnp.int32),
        src1=pltpu.VMEM((GROUP, CH), jnp.int32),
        dst0=pltpu.VMEM((GROUP, CH), jnp.int32),
        dst1=pltpu.VMEM((GROUP, CH), jnp.int32),
        val0=pltpu.VMEM((GROUP, CH), jnp.float32),
        val1=pltpu.VMEM((GROUP, CH), jnp.float32),
        rows0=pltpu.VMEM((GROUP * CH, D), jnp.float32),
        rows1=pltpu.VMEM((GROUP * CH, D), jnp.float32),
        stage_v=pltpu.VMEM((ZCH, D), jnp.float32),
        gsem0=pltpu.SemaphoreType.DMA,
        gsem1=pltpu.SemaphoreType.DMA,
        ssem0=pltpu.SemaphoreType.DMA,
        ssem1=pltpu.SemaphoreType.DMA,
        isem0=pltpu.SemaphoreType.DMA,
        isem1=pltpu.SemaphoreType.DMA,
    ),
)
def _propagate(x_hbm, src_hbm, dst_hbm, val_hbm, p0_hbm, p1_hbm,
               acc, src0, src1, dst0, dst1, val0, val1, rows0, rows1,
               stage_v, gsem0, gsem1, ssem0, ssem1, isem0, isem1):
    c = lax.axis_index("c")
    s = lax.axis_index("s")
    w = c * NS + s
    base_out = s * ROWS_PER_TILE
    row0 = w * R_PER_W

    idx_sets = ((src0, dst0, val0, gsem0, ssem0, isem0, rows0),
                (src1, dst1, val1, gsem1, ssem1, isem1, rows1))

    def _zrow(i, carry):
        stage_v[i, pl.ds(0, L)] = jnp.zeros((L,), jnp.float32)
        stage_v[i, pl.ds(L, L)] = jnp.zeros((L,), jnp.float32)
        return carry

    lax.fori_loop(0, ZCH, _zrow, 0)

    def _zchunk(k, carry):
        pltpu.sync_copy(stage_v, acc.at[pl.ds(base_out + k * ZCH, ZCH)])
        return carry

    lax.fori_loop(0, NZ, _zchunk, 0)
    plsc.subcore_barrier()

    def _issue_idx(g, S):
        src_v, dst_v, val_v, _, _, isem, _ = idx_sets[S]
        gr = row0 + g * GROUP
        pltpu.async_copy(src_hbm.at[pl.ds(gr, GROUP)], src_v, isem)
        pltpu.async_copy(dst_hbm.at[pl.ds(gr, GROUP)], dst_v, isem)
        pltpu.async_copy(val_hbm.at[pl.ds(gr, GROUP)], val_v, isem)

    def _wait_idx(g, S):
        src_v, dst_v, val_v, _, _, isem, _ = idx_sets[S]
        gr = row0 + g * GROUP
        pltpu.make_async_copy(src_hbm.at[pl.ds(gr, GROUP)], src_v, isem).wait()
        pltpu.make_async_copy(dst_hbm.at[pl.ds(gr, GROUP)], dst_v, isem).wait()
        pltpu.make_async_copy(val_hbm.at[pl.ds(gr, GROUP)], val_v, isem).wait()

    def _issue_gathers(S):
        src_v, _, _, gsem, _, _, rows_v = idx_sets[S]

        def _g(j, carry):
            pltpu.async_copy(x_hbm.at[src_v.at[j]],
                             rows_v.at[pl.ds(j * CH, CH)], gsem)
            return carry

        lax.fori_loop(0, GROUP, _g, 0)

    def _drain_scatters(S):
        _, dst_v, _, _, ssem, _, rows_v = idx_sets[S]

        def _d(j, carry):
            pltpu.make_async_copy(
                rows_v.at[pl.ds(j * CH, CH)], acc.at[dst_v.at[j]], ssem).wait()
            return carry

        lax.fori_loop(0, GROUP, _d, 0)

    def _process_group(S):
        src_v, dst_v, val_v, gsem, ssem, _, rows_v = idx_sets[S]

        def _chunk(j, carry):
            pltpu.make_async_copy(
                x_hbm.at[src_v.at[j]], rows_v.at[pl.ds(j * CH, CH)],
                gsem).wait()

            def _scale_blk(b, carry3):
                vals16 = val_v[j, pl.ds(b * L, L)]
                for l in range(L):
                    i = j * CH + b * L + l
                    v = vals16[l]
                    rows_v[i, pl.ds(0, L)] = rows_v[i, pl.ds(0, L)] * v
                    rows_v[i, pl.ds(L, L)] = rows_v[i, pl.ds(L, L)] * v
                return carry3

            lax.fori_loop(0, CH // L, _scale_blk, 0)
            pltpu.async_copy(rows_v.at[pl.ds(j * CH, CH)],
                             acc.at[dst_v.at[j]], ssem, add=True)
            return carry

        lax.fori_loop(0, GROUP, _chunk, 0)

    _issue_idx(0, 0)
    _wait_idx(0, 0)
    _issue_gathers(0)

    def _pair(t, carry):
        @pl.when(t >= 1)
        def _():
            _drain_scatters(1)
        _issue_idx(2 * t + 1, 1)
        _process_group(0)
        _wait_idx(2 * t + 1, 1)
        _issue_gathers(1)

        _drain_scatters(0)
        @pl.when(t < NPAIRS - 1)
        def _():
            _issue_idx(2 * t + 2, 0)
        _process_group(1)
        @pl.when(t < NPAIRS - 1)
        def _():
            _wait_idx(2 * t + 2, 0)
            _issue_gathers(0)
        return carry

    lax.fori_loop(0, NPAIRS, _pair, 0)
    _drain_scatters(1)
    plsc.subcore_barrier()

    def _dump(out_hbm):
        def _dchunk(k, carry):
            off = base_out + k * ZCH
            pltpu.sync_copy(acc.at[pl.ds(off, ZCH)], stage_v)
            pltpu.sync_copy(stage_v, out_hbm.at[pl.ds(off, ZCH)])
            return carry

        lax.fori_loop(0, NZ, _dchunk, 0)

    @pl.when(c == 0)
    def _():
        _dump(p0_hbm)

    @pl.when(c == 1)
    def _():
        _dump(p1_hbm)


def kernel(user_emb, item_emb, adj_indices, adj_values):
    all_emb = jnp.concatenate([user_emb, item_emb], axis=0)
    dkey = jax.random.key(12345)
    keep = jax.random.bernoulli(dkey, 1.0 - DROPOUT_P, all_emb.shape)
    x0 = jnp.where(keep, all_emb / (1.0 - DROPOUT_P), 0.0)

    idx = adj_indices.astype(jnp.int32)
    pad = E_PAD - E
    src_p = jnp.pad(idx[1], (0, pad)).reshape(-1, CH)
    dst_p = jnp.pad(idx[0], (0, pad)).reshape(-1, CH)
    val_p = jnp.pad(adj_values, (0, pad)).reshape(-1, CH)

    x = x0
    total = x0
    for _ in range(NUM_LAYERS):
        p0, p1 = _propagate(x, src_p, dst_p, val_p)
        x = (p0 + p1)[:N]
        total = total + x

    final = total * (1.0 / (NUM_LAYERS + 1))
    return final[:NUM_USERS], final[NUM_USERS:]


# speedup vs baseline: 2.7219x; 2.7219x over previous
import functools

import jax
import jax.numpy as jnp
import numpy as np
from jax import lax
from jax.experimental import pallas as pl
from jax.experimental.pallas import tpu as pltpu
from jax.experimental.pallas import tpu_sc as plsc

NUM_USERS = 25000
NUM_ITEMS = 25000
N = NUM_USERS + NUM_ITEMS
D = 32
HD = D // 2
E = 1600000
NUM_LAYERS = 3
DROPOUT_P = 0.2

NC, NS, L = 2, 16, 16
N_PAD = 50176
CH = 128
GROUP = 4
R_TOT = 12544
R_PER_TILE = R_TOT // NS
NGROUPS = R_PER_TILE // GROUP
NPAIRS = NGROUPS // 2
E_PAD = R_TOT * CH
ROWS_PER_TILE = N_PAD // NS
ZCH = 112
NZ = ROWS_PER_TILE // ZCH

_mesh = plsc.VectorSubcoreMesh(core_axis_name="c", subcore_axis_name="s")


def _dropout_bernoulli():
    key = jax.random.key(12345)
    return jax.random.bernoulli(key, 1.0 - DROPOUT_P, (N, D))


def _keep_mask():
    try:
        with jax.default_device(jax.local_devices(backend="cpu")[0]):
            return np.asarray(jax.jit(_dropout_bernoulli)())
    except Exception:
        pass
    try:
        return np.asarray(_dropout_bernoulli())
    except Exception:
        return None


_KEEP = _keep_mask()


@functools.partial(
    pl.kernel,
    out_type=jax.ShapeDtypeStruct((NC, N_PAD, HD), jnp.float32),
    mesh=_mesh,
    compiler_params=pltpu.CompilerParams(use_tc_tiling_on_sc=False),
    scratch_types=dict(
        xsp=pltpu.VMEM_SHARED((N_PAD, HD), jnp.float32),
        acc=pltpu.VMEM_SHARED((N_PAD, HD), jnp.float32),
        src0=pltpu.VMEM((GROUP, CH), jnp.int32),
        src1=pltpu.VMEM((GROUP, CH), jnp.int32),
        dst0=pltpu.VMEM((GROUP, CH), jnp.int32),
        dst1=pltpu.VMEM((GROUP, CH), jnp.int32),
        val0=pltpu.VMEM((GROUP, CH), jnp.float32),
        val1=pltpu.VMEM((GROUP, CH), jnp.float32),
        rows0=pltpu.VMEM((GROUP * CH, HD), jnp.float32),
        rows1=pltpu.VMEM((GROUP * CH, HD), jnp.float32),
        stage_v=pltpu.VMEM((ZCH, HD), jnp.float32),
        gsem0=pltpu.SemaphoreType.DMA,
        gsem1=pltpu.SemaphoreType.DMA,
        ssem0=pltpu.SemaphoreType.DMA,
        ssem1=pltpu.SemaphoreType.DMA,
        isem0=pltpu.SemaphoreType.DMA,
        isem1=pltpu.SemaphoreType.DMA,
    ),
)
def _propagate(x_hbm, src_hbm, dst_hbm, val_hbm, y_hbm,
               xsp, acc, src0, src1, dst0, dst1, val0, val1, rows0, rows1,
               stage_v, gsem0, gsem1, ssem0, ssem1, isem0, isem1):
    c = lax.axis_index("c")
    s = lax.axis_index("s")
    base_out = s * ROWS_PER_TILE
    row0 = s * R_PER_TILE

    idx_sets = ((src0, dst0, val0, gsem0, ssem0, isem0, rows0),
                (src1, dst1, val1, gsem1, ssem1, isem1, rows1))

    pltpu.sync_copy(x_hbm.at[c, pl.ds(base_out, ROWS_PER_TILE)],
                    xsp.at[pl.ds(base_out, ROWS_PER_TILE)])

    def _zrow(i, carry):
        stage_v[i, pl.ds(0, L)] = jnp.zeros((L,), jnp.float32)
        return carry

    lax.fori_loop(0, ZCH, _zrow, 0)

    def _zchunk(k, carry):
        pltpu.sync_copy(stage_v, acc.at[pl.ds(base_out + k * ZCH, ZCH)])
        return carry

    lax.fori_loop(0, NZ, _zchunk, 0)
    plsc.subcore_barrier()

    def _issue_idx(g, S):
        src_v, dst_v, val_v, _, _, isem, _ = idx_sets[S]
        gr = row0 + g * GROUP
        pltpu.async_copy(src_hbm.at[pl.ds(gr, GROUP)], src_v, isem)
        pltpu.async_copy(dst_hbm.at[pl.ds(gr, GROUP)], dst_v, isem)
        pltpu.async_copy(val_hbm.at[pl.ds(gr, GROUP)], val_v, isem)

    def _wait_idx(g, S):
        src_v, dst_v, val_v, _, _, isem, _ = idx_sets[S]
        gr = row0 + g * GROUP
        pltpu.make_async_copy(src_hbm.at[pl.ds(gr, GROUP)], src_v, isem).wait()
        pltpu.make_async_copy(dst_hbm.at[pl.ds(gr, GROUP)], dst_v, isem).wait()
        pltpu.make_async_copy(val_hbm.at[pl.ds(gr, GROUP)], val_v, isem).wait()

    def _issue_gathers(S):
        src_v, _, _, gsem, _, _, rows_v = idx_sets[S]

        def _g(j, carry):
            pltpu.async_copy(xsp.at[src_v.at[j]],
                             rows_v.at[pl.ds(j * CH, CH)], gsem)
            return carry

        lax.fori_loop(0, GROUP, _g, 0)

    def _drain_scatters(S):
        _, dst_v, _, _, ssem, _, rows_v = idx_sets[S]

        def _d(j, carry):
            pltpu.make_async_copy(
                rows_v.at[pl.ds(j * CH, CH)], acc.at[dst_v.at[j]], ssem).wait()
            return carry

        lax.fori_loop(0, GROUP, _d, 0)

    def _process_group(S):
        src_v, dst_v, val_v, gsem, ssem, _, rows_v = idx_sets[S]

        def _chunk(j, carry):
            pltpu.make_async_copy(
                xsp.at[src_v.at[j]], rows_v.at[pl.ds(j * CH, CH)],
                gsem).wait()

            def _scale_blk(b, carry3):
                vals16 = val_v[j, pl.ds(b * L, L)]
                for l in range(L):
                    i = j * CH + b * L + l
                    v = vals16[l]
                    rows_v[i, pl.ds(0, L)] = rows_v[i, pl.ds(0, L)] * v
                return carry3

            lax.fori_loop(0, CH // L, _scale_blk, 0)
            pltpu.async_copy(rows_v.at[pl.ds(j * CH, CH)],
                             acc.at[dst_v.at[j]], ssem, add=True)
            return carry

        lax.fori_loop(0, GROUP, _chunk, 0)

    _issue_idx(0, 0)
    _wait_idx(0, 0)
    _issue_gathers(0)

    def _pair(t, carry):
        @pl.when(t >= 1)
        def _():
            _drain_scatters(1)
        _issue_idx(2 * t + 1, 1)
        _process_group(0)
        _wait_idx(2 * t + 1, 1)
        _issue_gathers(1)

        _drain_scatters(0)
        @pl.when(t < NPAIRS - 1)
        def _():
            _issue_idx(2 * t + 2, 0)
        _process_group(1)
        @pl.when(t < NPAIRS - 1)
        def _():
            _wait_idx(2 * t + 2, 0)
            _issue_gathers(0)
        return carry

    lax.fori_loop(0, NPAIRS, _pair, 0)
    _drain_scatters(1)
    plsc.subcore_barrier()

    def _dchunk(k, carry):
        off = base_out + k * ZCH
        pltpu.sync_copy(acc.at[pl.ds(off, ZCH)], stage_v)
        pltpu.sync_copy(stage_v, y_hbm.at[c, pl.ds(off, ZCH)])
        return carry

    lax.fori_loop(0, NZ, _dchunk, 0)


def kernel(user_emb, item_emb, adj_indices, adj_values):
    all_emb = jnp.concatenate([user_emb, item_emb], axis=0)
    keep = jnp.asarray(_KEEP) if _KEEP is not None else _dropout_bernoulli()
    x0 = jnp.where(keep, all_emb / (1.0 - DROPOUT_P), 0.0)
    x0 = jnp.pad(x0, ((0, N_PAD - N), (0, 0)))

    idx = adj_indices.astype(jnp.int32)
    pad = E_PAD - E
    src_p = jnp.pad(idx[1], (0, pad)).reshape(-1, CH)
    dst_p = jnp.pad(idx[0], (0, pad)).reshape(-1, CH)
    val_p = jnp.pad(adj_values, (0, pad)).reshape(-1, CH)

    x2 = jnp.stack([x0[:, :HD], x0[:, HD:]])
    t2 = x2
    for _ in range(NUM_LAYERS):
        x2 = _propagate(x2, src_p, dst_p, val_p)
        t2 = t2 + x2

    final = jnp.concatenate([t2[0], t2[1]], axis=1)[:N] * (1.0 / (NUM_LAYERS + 1))
    return final[:NUM_USERS], final[NUM_USERS:]


# speedup vs baseline: 2.8649x; 1.0525x over previous
import functools

import jax
import jax.numpy as jnp
import numpy as np
from jax import lax
from jax.experimental import pallas as pl
from jax.experimental.pallas import tpu as pltpu
from jax.experimental.pallas import tpu_sc as plsc

NUM_USERS = 25000
NUM_ITEMS = 25000
N = NUM_USERS + NUM_ITEMS
D = 32
HD = D // 2
E = 1600000
NUM_LAYERS = 3
DROPOUT_P = 0.2

NC, NS, L = 2, 16, 16
N_PAD = 50176
CH = 128
GROUP = 4
R_TOT = 12544
R_PER_TILE = R_TOT // NS
NGROUPS = R_PER_TILE // GROUP
NPAIRS = NGROUPS // 2
E_PAD = R_TOT * CH
ROWS_PER_TILE = N_PAD // NS
ZCH = 112
NZ = ROWS_PER_TILE // ZCH

_mesh = plsc.VectorSubcoreMesh(core_axis_name="c", subcore_axis_name="s")


def _dropout_bernoulli():
    key = jax.random.key(12345)
    return jax.random.bernoulli(key, 1.0 - DROPOUT_P, (N, D))


def _keep_mask():
    try:
        with jax.default_device(jax.local_devices(backend="cpu")[0]):
            return np.asarray(jax.jit(_dropout_bernoulli)())
    except Exception:
        pass
    try:
        return np.asarray(_dropout_bernoulli())
    except Exception:
        return None


_KEEP = _keep_mask()


@functools.partial(
    pl.kernel,
    out_type=jax.ShapeDtypeStruct((NC, N_PAD, HD), jnp.float32),
    mesh=_mesh,
    compiler_params=pltpu.CompilerParams(use_tc_tiling_on_sc=False),
    scratch_types=dict(
        xsp=pltpu.VMEM_SHARED((N_PAD, HD), jnp.float32),
        acc=pltpu.VMEM_SHARED((N_PAD, HD), jnp.float32),
        src0=pltpu.VMEM((GROUP, CH), jnp.int32),
        src1=pltpu.VMEM((GROUP, CH), jnp.int32),
        dst0=pltpu.VMEM((GROUP, CH), jnp.int32),
        dst1=pltpu.VMEM((GROUP, CH), jnp.int32),
        val0=pltpu.VMEM((GROUP, CH), jnp.float32),
        val1=pltpu.VMEM((GROUP, CH), jnp.float32),
        rows0=pltpu.VMEM((GROUP * CH, HD), jnp.float32),
        rows1=pltpu.VMEM((GROUP * CH, HD), jnp.float32),
        stage_v=pltpu.VMEM((ZCH, HD), jnp.float32),
        gsem0=pltpu.SemaphoreType.DMA,
        gsem1=pltpu.SemaphoreType.DMA,
        ssem0=pltpu.SemaphoreType.DMA,
        ssem1=pltpu.SemaphoreType.DMA,
        isem0=pltpu.SemaphoreType.DMA,
        isem1=pltpu.SemaphoreType.DMA,
    ),
)
def _propagate(x_hbm, src_hbm, dst_hbm, val_hbm, y_hbm,
               xsp, acc, src0, src1, dst0, dst1, val0, val1, rows0, rows1,
               stage_v, gsem0, gsem1, ssem0, ssem1, isem0, isem1):
    c = lax.axis_index("c")
    s = lax.axis_index("s")
    base_out = s * ROWS_PER_TILE
    row0 = s * R_PER_TILE

    idx_sets = ((src0, dst0, val0, gsem0, ssem0, isem0, rows0),
                (src1, dst1, val1, gsem1, ssem1, isem1, rows1))

    pltpu.sync_copy(x_hbm.at[c, pl.ds(base_out, ROWS_PER_TILE)],
                    xsp.at[pl.ds(base_out, ROWS_PER_TILE)])

    def _zrow(i, carry):
        stage_v[i, pl.ds(0, L)] = jnp.zeros((L,), jnp.float32)
        return carry

    lax.fori_loop(0, ZCH, _zrow, 0)

    def _zchunk(k, carry):
        pltpu.sync_copy(stage_v, acc.at[pl.ds(base_out + k * ZCH, ZCH)])
        return carry

    lax.fori_loop(0, NZ, _zchunk, 0)
    plsc.subcore_barrier()

    def _issue_idx(g, S):
        src_v, dst_v, val_v, _, _, isem, _ = idx_sets[S]
        gr = row0 + g * GROUP
        pltpu.async_copy(src_hbm.at[pl.ds(gr, GROUP)], src_v, isem)
        pltpu.async_copy(dst_hbm.at[pl.ds(gr, GROUP)], dst_v, isem)
        pltpu.async_copy(val_hbm.at[pl.ds(gr, GROUP)], val_v, isem)

    def _wait_idx(g, S):
        src_v, dst_v, val_v, _, _, isem, _ = idx_sets[S]
        gr = row0 + g * GROUP
        pltpu.make_async_copy(src_hbm.at[pl.ds(gr, GROUP)], src_v, isem).wait()
        pltpu.make_async_copy(dst_hbm.at[pl.ds(gr, GROUP)], dst_v, isem).wait()
        pltpu.make_async_copy(val_hbm.at[pl.ds(gr, GROUP)], val_v, isem).wait()

    def _issue_gathers(S):
        src_v, _, _, gsem, _, _, rows_v = idx_sets[S]

        def _g(j, carry):
            pltpu.async_copy(xsp.at[src_v.at[j]],
                             rows_v.at[pl.ds(j * CH, CH)], gsem)
            return carry

        lax.fori_loop(0, GROUP, _g, 0)

    def _drain_scatters(S):
        _, dst_v, _, _, ssem, _, rows_v = idx_sets[S]

        def _d(j, carry):
            pltpu.make_async_copy(
                rows_v.at[pl.ds(j * CH, CH)], acc.at[dst_v.at[j]], ssem).wait()
            return carry

        lax.fori_loop(0, GROUP, _d, 0)

    def _process_group(S):
        src_v, dst_v, val_v, gsem, ssem, _, rows_v = idx_sets[S]

        def _chunk(j, carry):
            pltpu.make_async_copy(
                xsp.at[src_v.at[j]], rows_v.at[pl.ds(j * CH, CH)],
                gsem).wait()

            def _scale_blk(b, carry3):
                vals16 = val_v[j, pl.ds(b * L, L)]
                for l in range(L):
                    i = j * CH + b * L + l
                    v = vals16[l]
                    rows_v[i, pl.ds(0, L)] = rows_v[i, pl.ds(0, L)] * v
                return carry3

            lax.fori_loop(0, CH // L, _scale_blk, 0)
            pltpu.async_copy(rows_v.at[pl.ds(j * CH, CH)],
                             acc.at[dst_v.at[j]], ssem, add=True)
            return carry

        lax.fori_loop(0, GROUP, _chunk, 0)

    _issue_idx(0, 0)
    _wait_idx(0, 0)
    _issue_gathers(0)

    def _pair(t, carry):
        @pl.when(t >= 1)
        def _():
            _drain_scatters(1)
        _issue_idx(2 * t + 1, 1)
        _process_group(0)
        _wait_idx(2 * t + 1, 1)
        _issue_gathers(1)

        _drain_scatters(0)
        @pl.when(t < NPAIRS - 1)
        def _():
            _issue_idx(2 * t + 2, 0)
        _process_group(1)
        @pl.when(t < NPAIRS - 1)
        def _():
            _wait_idx(2 * t + 2, 0)
            _issue_gathers(0)
        return carry

    lax.fori_loop(0, NPAIRS, _pair, 0)
    _drain_scatters(1)
    plsc.subcore_barrier()

    def _dchunk(k, carry):
        off = base_out + k * ZCH
        pltpu.sync_copy(acc.at[pl.ds(off, ZCH)], stage_v)
        pltpu.sync_copy(stage_v, y_hbm.at[c, pl.ds(off, ZCH)])
        return carry

    lax.fori_loop(0, NZ, _dchunk, 0)


RCH = 784
NRZ = ROWS_PER_TILE // RCH


@functools.partial(
    pl.kernel,
    out_type=jax.ShapeDtypeStruct((NC, N_PAD, HD), jnp.float32),
    mesh=_mesh,
    compiler_params=pltpu.CompilerParams(use_tc_tiling_on_sc=False),
    scratch_types=dict(
        b0=pltpu.VMEM((RCH, HD), jnp.float32),
        b1=pltpu.VMEM((RCH, HD), jnp.float32),
        b2=pltpu.VMEM((RCH, HD), jnp.float32),
        b3=pltpu.VMEM((RCH, HD), jnp.float32),
        rsem=pltpu.SemaphoreType.DMA,
    ),
)
def _mean4(x0_hbm, y1_hbm, y2_hbm, y3_hbm, f_hbm, b0, b1, b2, b3, rsem):
    c = lax.axis_index("c")
    s = lax.axis_index("s")
    base = s * ROWS_PER_TILE

    def _chunk(k, carry):
        off = base + k * RCH
        pltpu.async_copy(x0_hbm.at[c, pl.ds(off, RCH)], b0, rsem)
        pltpu.async_copy(y1_hbm.at[c, pl.ds(off, RCH)], b1, rsem)
        pltpu.async_copy(y2_hbm.at[c, pl.ds(off, RCH)], b2, rsem)
        pltpu.async_copy(y3_hbm.at[c, pl.ds(off, RCH)], b3, rsem)
        pltpu.make_async_copy(x0_hbm.at[c, pl.ds(off, RCH)], b0, rsem).wait()
        pltpu.make_async_copy(y1_hbm.at[c, pl.ds(off, RCH)], b1, rsem).wait()
        pltpu.make_async_copy(y2_hbm.at[c, pl.ds(off, RCH)], b2, rsem).wait()
        pltpu.make_async_copy(y3_hbm.at[c, pl.ds(off, RCH)], b3, rsem).wait()

        def _row(i, carry2):
            acc16 = ((b0[i, pl.ds(0, L)] + b1[i, pl.ds(0, L)])
                     + (b2[i, pl.ds(0, L)] + b3[i, pl.ds(0, L)]))
            b0[i, pl.ds(0, L)] = acc16 * 0.25
            return carry2

        lax.fori_loop(0, RCH, _row, 0)
        pltpu.sync_copy(b0, f_hbm.at[c, pl.ds(off, RCH)])
        return carry

    lax.fori_loop(0, NRZ, _chunk, 0)


def kernel(user_emb, item_emb, adj_indices, adj_values):
    all_emb = jnp.concatenate([user_emb, item_emb], axis=0)
    keep = jnp.asarray(_KEEP) if _KEEP is not None else _dropout_bernoulli()
    x0 = jnp.where(keep, all_emb / (1.0 - DROPOUT_P), 0.0)
    x0 = jnp.pad(x0, ((0, N_PAD - N), (0, 0)))

    idx = adj_indices.astype(jnp.int32)
    pad = E_PAD - E
    src_p = jnp.pad(idx[1], (0, pad)).reshape(-1, CH)
    dst_p = jnp.pad(idx[0], (0, pad)).reshape(-1, CH)
    val_p = jnp.pad(adj_values, (0, pad)).reshape(-1, CH)

    x2 = jnp.stack([x0[:, :HD], x0[:, HD:]])
    ys = []
    y = x2
    for _ in range(NUM_LAYERS):
        y = _propagate(y, src_p, dst_p, val_p)
        ys.append(y)

    f2 = _mean4(x2, ys[0], ys[1], ys[2])
    final = jnp.concatenate([f2[0], f2[1]], axis=1)[:N]
    return final[:NUM_USERS], final[NUM_USERS:]
